# baseline (speedup 1.0000x reference)
import functools

import jax
import jax.numpy as jnp
from jax import lax
from jax.experimental import pallas as pl

N_EXPERTS = 64
TOP_K = 8
N_TOKENS = 32768
BLOCK_R = 4096
GRID = N_TOKENS // BLOCK_R


def _router_body(x_ref, w_ref, ids_ref, hist_ref):
    x = x_ref[...].T
    iota0 = lax.broadcasted_iota(jnp.int32, (N_EXPERTS, BLOCK_R), 0)
    neg_inf = jnp.float32(-jnp.inf)

    vals = []
    ids = []
    for _ in range(TOP_K):
        m = jnp.max(x, axis=0, keepdims=True)
        cand = jnp.where(x == m, iota0, N_EXPERTS)
        idx = jnp.min(cand, axis=0, keepdims=True)
        vals.append(m)
        ids.append(idx)
        x = jnp.where(iota0 == idx, neg_inf, x)

    v8 = jnp.concatenate(vals, axis=0)
    i8 = jnp.concatenate(ids, axis=0)
    e = jnp.exp(v8 - v8[0:1, :])
    w_ref[...] = e / jnp.sum(e, axis=0, keepdims=True)
    ids_ref[...] = i8

    sel = jnp.where(x == neg_inf, 1.0, 0.0)
    partial = jnp.sum(sel, axis=1, keepdims=True)
    @pl.when(pl.program_id(0) == 0)
    def _():
        hist_ref[...] = jnp.zeros_like(hist_ref)
    hist_ref[...] += partial


@functools.partial(jax.jit)
def kernel(logits):
    w8, ids8, hist = pl.pallas_call(
        _router_body,
        grid=(GRID,),
        in_specs=[pl.BlockSpec((BLOCK_R, N_EXPERTS), lambda i: (i, 0))],
        out_specs=[
            pl.BlockSpec((TOP_K, BLOCK_R), lambda i: (0, i)),
            pl.BlockSpec((TOP_K, BLOCK_R), lambda i: (0, i)),
            pl.BlockSpec((N_EXPERTS, 1), lambda i: (0, 0)),
        ],
        out_shape=[
            jax.ShapeDtypeStruct((TOP_K, N_TOKENS), jnp.float32),
            jax.ShapeDtypeStruct((TOP_K, N_TOKENS), jnp.int32),
            jax.ShapeDtypeStruct((N_EXPERTS, 1), jnp.float32),
        ],
    )(logits)
    return (logits, w8.T, ids8.T, hist.reshape(N_EXPERTS))
